# Initial kernel scaffold; baseline (speedup 1.0000x reference)
import jax
import jax.numpy as jnp
from jax.experimental import pallas as pl


def kernel(x, ix, Wf, bf, Wg, bg, Wh, bh):
    raise NotImplementedError("write your pallas kernel here")


# speedup vs baseline: 8.8196x; 8.8196x over previous
import functools

import jax
import jax.numpy as jnp
from jax import lax
from jax.experimental import pallas as pl
from jax.experimental.pallas import tpu as pltpu
from jax.experimental.pallas import tpu_sc as plsc

N = 320000
D = 128
NSEG = 10000

CHUNK = 128
CHUNKS_PER_SUB = 157
NCHUNK = 16 * CHUNKS_PER_SUB
NPAD = NCHUNK * CHUNK
B1 = 2048
SPAD = 10112
ZROWS = SPAD // 16
GCHUNK = 400
GITER = 10000 // GCHUNK

_mesh = plsc.VectorSubcoreMesh(core_axis_name="c", subcore_axis_name="s")


def _k1_body(x_ref, wf_ref, bf_ref, wg_ref, bg_ref, o_ref):
    xb = x_ref[...]
    fx = jnp.dot(xb, wf_ref[...], preferred_element_type=jnp.float32) + bf_ref[...]
    gx = jnp.dot(xb, wg_ref[...], preferred_element_type=jnp.float32) + bg_ref[...]
    e = jnp.exp(jnp.clip(gx, -50.0, 50.0))
    o_ref[0] = fx * e
    o_ref[1] = e


_k1 = pl.pallas_call(
    _k1_body,
    grid=(NPAD // B1,),
    in_specs=[
        pl.BlockSpec((B1, D), lambda i: (i, 0)),
        pl.BlockSpec((D, D), lambda i: (0, 0)),
        pl.BlockSpec((1, D), lambda i: (0, 0)),
        pl.BlockSpec((D, D), lambda i: (0, 0)),
        pl.BlockSpec((1, D), lambda i: (0, 0)),
    ],
    out_specs=pl.BlockSpec((2, B1, D), lambda i: (0, i, 0)),
    out_shape=jax.ShapeDtypeStruct((2, NPAD, D), jnp.float32),
)


@functools.partial(
    pl.kernel,
    mesh=_mesh,
    out_type=jax.ShapeDtypeStruct((2, SPAD, D), jnp.float32),
    scratch_types=[
        pltpu.VMEM((CHUNKS_PER_SUB, CHUNK), jnp.int32),
        pltpu.VMEM((CHUNK, D), jnp.float32),
        pltpu.VMEM_SHARED((SPAD, D), jnp.float32),
    ],
)
def _k2(contrib_hbm, ix3d_hbm, zeros_hbm, out_hbm, idx_v, val_v, acc):
    c = lax.axis_index("c")
    s = lax.axis_index("s")
    pltpu.sync_copy(zeros_hbm, acc.at[pl.ds(s * ZROWS, ZROWS)])
    pltpu.sync_copy(ix3d_hbm.at[s], idx_v)
    plsc.subcore_barrier()

    def body(j, carry):
        row0 = (s * CHUNKS_PER_SUB + j) * CHUNK
        pltpu.sync_copy(contrib_hbm.at[c, pl.ds(row0, CHUNK)], val_v)
        pltpu.sync_copy(val_v, acc.at[idx_v.at[j]], add=True)
        return carry

    lax.fori_loop(0, CHUNKS_PER_SUB, body, 0)
    plsc.subcore_barrier()
    pltpu.sync_copy(
        acc.at[pl.ds(s * ZROWS, ZROWS)], out_hbm.at[c, pl.ds(s * ZROWS, ZROWS)]
    )


def _k3_body(num_ref, den_ref, wh_ref, bh_ref, o_ref):
    y = num_ref[0] / den_ref[0]
    o_ref[...] = jnp.dot(y, wh_ref[...], preferred_element_type=jnp.float32) + bh_ref[...]


_k3 = pl.pallas_call(
    _k3_body,
    grid=(8,),
    in_specs=[
        pl.BlockSpec((1, SPAD // 8, D), lambda i: (0, i, 0)),
        pl.BlockSpec((1, SPAD // 8, D), lambda i: (1, i, 0)),
        pl.BlockSpec((D, D), lambda i: (0, 0)),
        pl.BlockSpec((1, D), lambda i: (0, 0)),
    ],
    out_specs=pl.BlockSpec((SPAD // 8, D), lambda i: (i, 0)),
    out_shape=jax.ShapeDtypeStruct((SPAD, D), jnp.float32),
)


@functools.partial(
    pl.kernel,
    mesh=_mesh,
    out_type=jax.ShapeDtypeStruct((N, D), jnp.float32),
    scratch_types=[
        pltpu.VMEM((GCHUNK,), jnp.int32),
        pltpu.VMEM((GCHUNK, D), jnp.float32),
        pltpu.SemaphoreType.DMA,
    ],
)
def _k4(yh_hbm, ix_hbm, out_hbm, idx_v, rows_v, sem):
    c = lax.axis_index("c")
    s = lax.axis_index("s")
    base = (s * 2 + c) * (N // 32)

    def body(j, carry):
        off = base + j * GCHUNK
        pltpu.sync_copy(ix_hbm.at[pl.ds(off, GCHUNK)], idx_v)
        pltpu.async_copy(yh_hbm.at[idx_v], rows_v, sem).wait()
        pltpu.sync_copy(rows_v, out_hbm.at[pl.ds(off, GCHUNK)])
        return carry

    lax.fori_loop(0, GITER, body, 0)


def kernel(x, ix, Wf, bf, Wg, bg, Wh, bh):
    x2d = x.reshape(N, D)
    ix_i = ix.reshape(N).astype(jnp.int32)
    contrib = _k1(x2d, Wf.T, bf[None], Wg.T, bg[None])
    ix_pad = (
        jnp.full((NPAD,), NSEG, jnp.int32)
        .at[:N]
        .set(ix_i)
        .reshape(16, CHUNKS_PER_SUB, CHUNK)
    )
    zeros = jnp.zeros((ZROWS, D), jnp.float32)
    sums = _k2(contrib, ix_pad, zeros)
    yh = _k3(sums, sums, Wh.T, bh[None])
    out = _k4(yh, ix_i)
    return out[None]
